# baseline (speedup 1.0000x reference)
import functools

import jax
import jax.numpy as jnp
from jax import lax
from jax.experimental import pallas as pl
from jax.experimental.pallas import tpu as pltpu
from jax.experimental.pallas import tpu_sc as plsc

_BATCH = 64
_TOKENS = 128
_DIM = 4096
_ROWS = _BATCH * _TOKENS

_NC = 2
_NS = 16
_NW = _NC * _NS
_B_PER_W = _ROWS // _NW
_STAGE = _TOKENS // _NS


def _make_sc_lookup():
    mesh = plsc.VectorSubcoreMesh(core_axis_name="c", subcore_axis_name="s")

    @functools.partial(
        pl.kernel,
        mesh=mesh,
        out_type=jax.ShapeDtypeStruct((_ROWS, _DIM), jnp.float32),
        scratch_types=[
            pltpu.VMEM((_B_PER_W,), jnp.int32),
            pltpu.VMEM_SHARED((_TOKENS, _DIM), jnp.float32),
            pltpu.SemaphoreType.DMA,
        ],
    )
    def sc_lookup(idx_hbm, table_hbm, out_hbm, idx_v, shared_tab, wsem):
        sid = lax.axis_index("s")
        wid = sid * _NC + lax.axis_index("c")
        base = wid * _B_PER_W
        pltpu.sync_copy(table_hbm.at[pl.ds(sid * _STAGE, _STAGE)],
                        shared_tab.at[pl.ds(sid * _STAGE, _STAGE)])
        pltpu.sync_copy(idx_hbm.at[pl.ds(base, _B_PER_W)], idx_v)
        plsc.subcore_barrier()

        def issue(g, carry):
            vec = idx_v[pl.ds(g * 16, 16)]
            for k in range(16):
                pltpu.async_copy(
                    shared_tab.at[pl.ds(vec[k], 1)],
                    out_hbm.at[pl.ds(base + g * 16 + k, 1)], wsem)
            return carry

        lax.fori_loop(0, _B_PER_W // 16, issue, 0)

        def drain(k, carry):
            pltpu.make_async_copy(
                shared_tab.at[pl.ds(0, 1)],
                out_hbm.at[pl.ds(base, 1)], wsem).wait()
            return carry

        lax.fori_loop(0, _B_PER_W, drain, 0)

    return sc_lookup


_sc_lookup = _make_sc_lookup()


def kernel(indices, table):
    idx_flat = indices.reshape(_ROWS).astype(jnp.int32)
    out = _sc_lookup(idx_flat, table)
    return out.reshape(_BATCH, _TOKENS, _DIM)


# speedup vs baseline: 1.3030x; 1.3030x over previous
import functools

import jax
import jax.numpy as jnp
from jax import lax
from jax.experimental import pallas as pl
from jax.experimental.pallas import tpu as pltpu
from jax.experimental.pallas import tpu_sc as plsc

_BATCH = 64
_TOKENS = 128
_DIM = 4096
_ROWS = _BATCH * _TOKENS

_NC = 2
_NS = 16
_NW = _NC * _NS
_B_PER_W = _ROWS // _NW
_CH = 8
_NPAIR = _B_PER_W // (2 * _CH)
_STAGE = _TOKENS // _NS


def _make_sc_lookup():
    mesh = plsc.VectorSubcoreMesh(core_axis_name="c", subcore_axis_name="s")

    @functools.partial(
        pl.kernel,
        mesh=mesh,
        out_type=jax.ShapeDtypeStruct((_ROWS, _DIM), jnp.float32),
        scratch_types=[
            pltpu.VMEM((_B_PER_W,), jnp.int32),
            pltpu.VMEM((2, _CH, _DIM), jnp.float32),
            pltpu.VMEM_SHARED((_TOKENS, _DIM), jnp.float32),
            pltpu.SemaphoreType.DMA,
            pltpu.SemaphoreType.DMA,
            pltpu.SemaphoreType.DMA,
            pltpu.SemaphoreType.DMA,
            pltpu.SemaphoreType.DMA,
        ],
    )
    def sc_lookup(idx_hbm, table_hbm, out_hbm, idx_v, bufs, shared_tab,
                  csem0, csem1, wsem0, wsem1, dsem):
        sid = lax.axis_index("s")
        wid = sid * _NC + lax.axis_index("c")
        base = wid * _B_PER_W
        pltpu.sync_copy(table_hbm.at[pl.ds(sid * _STAGE, _STAGE)],
                        shared_tab.at[pl.ds(sid * _STAGE, _STAGE)])
        pltpu.sync_copy(idx_hbm.at[pl.ds(base, _B_PER_W)], idx_v)
        plsc.subcore_barrier()

        csems = (csem0, csem1)
        wsems = (wsem0, wsem1)

        def do_pair(p, buf):
            vec = idx_v[pl.ds(p * 16, 16)]
            handles = []
            for k in range(_CH):
                handles.append(pltpu.async_copy(
                    shared_tab.at[vec[k]], bufs.at[buf].at[k], csems[buf]))
            for k in range(_CH):
                pltpu.async_copy(
                    shared_tab.at[pl.ds(vec[_CH + k], 1)],
                    out_hbm.at[pl.ds(base + p * 16 + _CH + k, 1)], dsem)
            for h in handles:
                h.wait()
            pltpu.async_copy(
                bufs.at[buf], out_hbm.at[pl.ds(base + p * 16, _CH)],
                wsems[buf])

        def wait_write(p, buf):
            pltpu.make_async_copy(
                bufs.at[buf], out_hbm.at[pl.ds(base + p * 16, _CH)],
                wsems[buf]).wait()

        do_pair(0, 0)
        do_pair(1, 1)

        def step(i, carry):
            for u in range(2):
                p = 2 + i * 2 + u
                wait_write(p - 2, u)
                do_pair(p, u)
            return carry

        lax.fori_loop(0, (_NPAIR - 2) // 2, step, 0)
        wait_write(_NPAIR - 2, 0)
        wait_write(_NPAIR - 1, 1)

        def drain(k, carry):
            pltpu.make_async_copy(
                shared_tab.at[pl.ds(0, 1)],
                out_hbm.at[pl.ds(base, 1)], dsem).wait()
            return carry

        lax.fori_loop(0, _NPAIR * _CH, drain, 0)

    return sc_lookup


_sc_lookup = _make_sc_lookup()


def kernel(indices, table):
    idx_flat = indices.reshape(_ROWS).astype(jnp.int32)
    out = _sc_lookup(idx_flat, table)
    return out.reshape(_BATCH, _TOKENS, _DIM)
